# Initial kernel scaffold; baseline (speedup 1.0000x reference)
import jax
import jax.numpy as jnp
from jax.experimental import pallas as pl


def kernel(data, iteration):
    raise NotImplementedError("write your pallas kernel here")


# speedup vs baseline: 5.3982x; 5.3982x over previous
import jax
import jax.numpy as jnp
from jax.experimental import pallas as pl
from jax.experimental.pallas import tpu as pltpu

_B, _N, _F, _K = 4, 4096, 64, 64


def _init_ids():
    keys = jax.random.split(jax.random.key(42), _B)
    ids = jnp.stack(
        [jax.random.permutation(kk, _N)[:_K] for kk in keys], axis=0
    )
    return ids.astype(jnp.int32)


def _argmin_axis0(score):
    k = score.shape[0]
    s = jnp.where(jnp.isnan(score), -jnp.inf, score)
    minv = jnp.min(s, axis=0, keepdims=True)
    iota = jax.lax.broadcasted_iota(jnp.int32, s.shape, 0)
    return jnp.min(jnp.where(s == minv, iota, k), axis=0).astype(jnp.int32)


def _kmeans_kernel(it_ref, ids_ref, data_ref, out_ref, cent_ref):
    b = pl.program_id(0)
    data = data_ref[0]

    for k in range(_K):
        cent_ref[pl.ds(k, 1), :] = data_ref[0, pl.ds(ids_ref[b, k], 1), :]

    def assign(cents):
        c_sq = jnp.sum(cents * cents, axis=1, keepdims=True)
        cross = jax.lax.dot_general(
            cents, data, (((1,), (1,)), ((), ())),
            precision=jax.lax.Precision.HIGHEST,
        )
        score = c_sq - 2.0 * cross
        return _argmin_axis0(score)

    def body(_, cid):
        onehot = (
            jax.lax.broadcasted_iota(jnp.int32, (_K, _N), 0) == cid[None, :]
        ).astype(jnp.float32)
        counts = jnp.sum(onehot, axis=1, keepdims=True)
        sums = jax.lax.dot_general(
            onehot, data, (((1,), (0,)), ((), ())),
            precision=jax.lax.Precision.HIGHEST,
        )
        return assign(sums / counts)

    cid = jax.lax.fori_loop(0, it_ref[0], body, assign(cent_ref[...]))
    out_ref[0, 0, :] = cid


def kernel(data, iteration):
    it = jnp.asarray(iteration, dtype=jnp.int32).reshape((1,))
    ids = _init_ids()
    out = pl.pallas_call(
        _kmeans_kernel,
        grid=(_B,),
        in_specs=[
            pl.BlockSpec(memory_space=pltpu.SMEM),
            pl.BlockSpec(memory_space=pltpu.SMEM),
            pl.BlockSpec((1, _N, _F), lambda b: (b, 0, 0)),
        ],
        out_specs=pl.BlockSpec((1, 1, _N), lambda b: (b, 0, 0)),
        out_shape=jax.ShapeDtypeStruct((_B, 1, _N), jnp.int32),
        scratch_shapes=[pltpu.VMEM((_K, _F), jnp.float32)],
        compiler_params=pltpu.CompilerParams(
            dimension_semantics=("parallel",),
        ),
    )(it, ids, data)
    return out.reshape(_B, _N)
